# baseline (speedup 1.0000x reference)
import functools

import jax
import jax.numpy as jnp
from jax import lax
from jax.experimental import pallas as pl
from jax.experimental.pallas import tpu as pltpu
from jax.experimental.pallas import tpu_sc as plsc

N_NODES = 10000
N_PAD = 10240
N_EDGES = 320000
NW = 32
N_CHUNK = 80
CHUNK = 128
EPW = N_EDGES // NW
EPT = N_CHUNK * CHUNK
PADE = EPT - EPW
ROWS_PER_TILE = N_PAD // 16
FEAT = 128


def _sc_edge_agg(table, src_flat, dst3, zeros_hbm):
    mesh = plsc.VectorSubcoreMesh(core_axis_name="c", subcore_axis_name="s")

    @functools.partial(
        pl.kernel,
        out_type=jax.ShapeDtypeStruct((2, N_PAD, FEAT), jnp.float32),
        mesh=mesh,
        scratch_types=[
            pltpu.VMEM((N_CHUNK, CHUNK), jnp.int32),
            pltpu.VMEM((N_CHUNK, CHUNK), jnp.int32),
            pltpu.VMEM((CHUNK, FEAT), jnp.float32),
            pltpu.VMEM_SHARED((N_PAD, FEAT), jnp.float32),
            pltpu.SemaphoreType.DMA,
            pltpu.SemaphoreType.DMA,
        ],
    )
    def k(table_h, src_h, dst_h, zeros_h, out_h,
          src_v, dst_v, rows_v, acc_sh, sem_a, sem_b):
        c = lax.axis_index("c")
        s = lax.axis_index("s")
        w = c * 16 + s
        pltpu.sync_copy(src_h.at[w], src_v)
        pltpu.sync_copy(dst_h.at[w], dst_v)
        pltpu.sync_copy(zeros_h, acc_sh.at[pl.ds(s * ROWS_PER_TILE, ROWS_PER_TILE)])
        plsc.subcore_barrier()

        def body(p, carry):
            pltpu.sync_copy(table_h.at[src_v.at[p]], rows_v)
            pltpu.sync_copy(rows_v, acc_sh.at[dst_v.at[p]], add=True)
            return carry

        lax.fori_loop(0, N_CHUNK, body, 0)
        plsc.subcore_barrier()
        sl = pl.ds(s * ROWS_PER_TILE, ROWS_PER_TILE)
        pltpu.sync_copy(acc_sh.at[sl], out_h.at[c].at[sl])

    return k(table, src_flat, dst3, zeros_hbm)


def _sc_degree(dst3, ones_hbm, zeros_hbm):
    mesh = plsc.VectorSubcoreMesh(core_axis_name="c", subcore_axis_name="s")

    @functools.partial(
        pl.kernel,
        out_type=jax.ShapeDtypeStruct((2, N_PAD, FEAT), jnp.float32),
        mesh=mesh,
        scratch_types=[
            pltpu.VMEM((N_CHUNK, CHUNK), jnp.int32),
            pltpu.VMEM((CHUNK, FEAT), jnp.float32),
            pltpu.VMEM_SHARED((N_PAD, FEAT), jnp.float32),
            pltpu.SemaphoreType.DMA,
            pltpu.SemaphoreType.DMA,
        ],
    )
    def k(dst_h, ones_h, zeros_h, out_h, dst_v, ones_v, acc_sh, sem_a, sem_b):
        c = lax.axis_index("c")
        s = lax.axis_index("s")
        w = c * 16 + s
        pltpu.sync_copy(dst_h.at[w], dst_v)
        pltpu.sync_copy(ones_h, ones_v)
        pltpu.sync_copy(zeros_h, acc_sh.at[pl.ds(s * ROWS_PER_TILE, ROWS_PER_TILE)])
        plsc.subcore_barrier()

        def body(p, carry):
            pltpu.sync_copy(ones_v, acc_sh.at[dst_v.at[p]], add=True)
            return carry

        lax.fori_loop(0, N_CHUNK, body, 0)
        plsc.subcore_barrier()
        sl = pl.ds(s * ROWS_PER_TILE, ROWS_PER_TILE)
        pltpu.sync_copy(acc_sh.at[sl], out_h.at[c].at[sl])

    return k(dst3, ones_hbm, zeros_hbm)


_R = 1000


def _part_specs():
    return [
        pl.BlockSpec((1, _R, FEAT), lambda i: (0, i, 0)),
        pl.BlockSpec((1, _R, FEAT), lambda i: (1, i, 0)),
    ]


def _tc_encode1(x, W1, degp):
    def body(x_r, w_r, d0_r, d1_r, h_r, dis_r):
        deg = 1.0 + d0_r[0][:, 0:1] + d1_r[0][:, 0:1]
        dis = lax.rsqrt(deg)
        h = jnp.dot(x_r[...], w_r[...], preferred_element_type=jnp.float32,
                    precision=lax.Precision.HIGHEST)
        h_r[...] = h * dis
        dis_r[...] = dis

    return pl.pallas_call(
        body,
        grid=(N_NODES // _R,),
        in_specs=[
            pl.BlockSpec((_R, 128), lambda i: (i, 0)),
            pl.BlockSpec((128, 128), lambda i: (0, 0)),
        ] + _part_specs(),
        out_specs=[
            pl.BlockSpec((_R, 128), lambda i: (i, 0)),
            pl.BlockSpec((_R, 1), lambda i: (i, 0)),
        ],
        out_shape=[
            jax.ShapeDtypeStruct((N_NODES, 128), jnp.float32),
            jax.ShapeDtypeStruct((N_NODES, 1), jnp.float32),
        ],
    )(x, W1, degp, degp)


def _tc_encode2(aggp, h1p, dis, b1, W2):
    def body(a0_r, a1_r, h1p_r, dis_r, b1_r, w2_r, o_r):
        t = (a0_r[0] + a1_r[0] + h1p_r[...]) * dis_r[...] + b1_r[...]
        o = jnp.maximum(t, 0.0)
        h2 = jnp.dot(o, w2_r[...], preferred_element_type=jnp.float32,
                     precision=lax.Precision.HIGHEST)
        o_r[...] = jnp.concatenate(
            [h2 * dis_r[...], jnp.zeros((h2.shape[0], 64), jnp.float32)], axis=1)

    return pl.pallas_call(
        body,
        grid=(N_NODES // _R,),
        in_specs=_part_specs() + [
            pl.BlockSpec((_R, 128), lambda i: (i, 0)),
            pl.BlockSpec((_R, 1), lambda i: (i, 0)),
            pl.BlockSpec((1, 128), lambda i: (0, 0)),
            pl.BlockSpec((128, 64), lambda i: (0, 0)),
        ],
        out_specs=pl.BlockSpec((_R, 128), lambda i: (i, 0)),
        out_shape=jax.ShapeDtypeStruct((N_NODES, 128), jnp.float32),
    )(aggp, aggp, h1p, dis, b1, W2)


def _tc_decode(aggp, h2p, dis, b2p, Wdp, bd):
    def body(a0_r, a1_r, h2p_r, dis_r, b2_r, wd_r, bd_r, o_r):
        z = (a0_r[0] + a1_r[0] + h2p_r[...]) * dis_r[...] + b2_r[...]
        dec = jnp.dot(z, wd_r[...], preferred_element_type=jnp.float32,
                      precision=lax.Precision.HIGHEST) + bd_r[...]
        o_r[...] = 1.0 / (1.0 + jnp.exp(-dec))

    return pl.pallas_call(
        body,
        grid=(N_NODES // _R,),
        in_specs=_part_specs() + [
            pl.BlockSpec((_R, 128), lambda i: (i, 0)),
            pl.BlockSpec((_R, 1), lambda i: (i, 0)),
            pl.BlockSpec((1, 128), lambda i: (0, 0)),
            pl.BlockSpec((128, 128), lambda i: (0, 0)),
            pl.BlockSpec((1, 128), lambda i: (0, 0)),
        ],
        out_specs=pl.BlockSpec((_R, 128), lambda i: (i, 0)),
        out_shape=jax.ShapeDtypeStruct((N_NODES, 128), jnp.float32),
    )(aggp, aggp, h2p, dis, b2p, Wdp, bd)


def kernel(x, edge_index, W1, b1, W2, b2, Wd, bd):
    srcw = edge_index[0].reshape(NW, EPW)
    dstw = edge_index[1].reshape(NW, EPW)
    pad_s = jnp.zeros((NW, PADE), jnp.int32)
    pad_d = jnp.broadcast_to(N_NODES + jnp.arange(PADE, dtype=jnp.int32),
                             (NW, PADE))
    src3 = jnp.concatenate([srcw, pad_s], axis=1).reshape(NW, N_CHUNK, CHUNK)
    dst3 = jnp.concatenate([dstw, pad_d], axis=1).reshape(NW, N_CHUNK, CHUNK)

    zeros128 = jnp.zeros((ROWS_PER_TILE, FEAT), jnp.float32)
    ones128 = jnp.ones((CHUNK, FEAT), jnp.float32)
    degp = _sc_degree(dst3, ones128, zeros128)

    h1p, dis = _tc_encode1(x, W1, degp)

    agg1 = _sc_edge_agg(h1p, src3, dst3, zeros128)
    h2p = _tc_encode2(agg1, h1p, dis, b1.reshape(1, 128), W2)

    agg2 = _sc_edge_agg(h2p, src3, dst3, zeros128)
    b2p = jnp.zeros((1, 128), jnp.float32).at[0, :64].set(b2)
    Wdp = jnp.zeros((128, 128), jnp.float32).at[:64, :].set(Wd)
    return _tc_decode(agg2, h2p, dis, b2p, Wdp, bd.reshape(1, 128))


# speedup vs baseline: 1.2092x; 1.2092x over previous
import functools

import jax
import jax.numpy as jnp
from jax import lax
from jax.experimental import pallas as pl
from jax.experimental.pallas import tpu as pltpu
from jax.experimental.pallas import tpu_sc as plsc

N_NODES = 10000
N_PAD = 10240
N_EDGES = 320000
NW = 32
N_CHUNK = 53
CHUNK = 192
EPW = N_EDGES // NW
EPT = N_CHUNK * CHUNK
PADE = EPT - EPW
ROWS_PER_TILE = N_PAD // 16
FEAT = 128


def _sc_edge_agg(table, src_flat, dst3, zeros_hbm):
    mesh = plsc.VectorSubcoreMesh(core_axis_name="c", subcore_axis_name="s")

    @functools.partial(
        pl.kernel,
        out_type=jax.ShapeDtypeStruct((2, N_PAD, FEAT), jnp.float32),
        mesh=mesh,
        scratch_types=[
            pltpu.VMEM((EPT,), jnp.int32),
            pltpu.VMEM((EPT,), jnp.int32),
            pltpu.VMEM((CHUNK, FEAT), jnp.float32),
            pltpu.VMEM_SHARED((N_PAD, FEAT), jnp.float32),
            pltpu.SemaphoreType.DMA,
            pltpu.SemaphoreType.DMA,
        ],
    )
    def k(table_h, src_h, dst_h, zeros_h, out_h,
          src_v, dst_v, rows_v, acc_sh, sem_a, sem_b):
        c = lax.axis_index("c")
        s = lax.axis_index("s")
        w = c * 16 + s
        pltpu.sync_copy(src_h.at[pl.ds(w * EPT, EPT)], src_v)
        pltpu.sync_copy(dst_h.at[pl.ds(w * EPT, EPT)], dst_v)
        pltpu.sync_copy(zeros_h, acc_sh.at[pl.ds(s * ROWS_PER_TILE, ROWS_PER_TILE)])
        plsc.subcore_barrier()

        def body(p, carry):
            src_sl = src_v.at[pl.ds(p * CHUNK, CHUNK)]
            pltpu.async_copy(table_h.at[src_sl], rows_v, sem_a).wait()
            pltpu.sync_copy(rows_v, acc_sh.at[dst_v.at[pl.ds(p * CHUNK, CHUNK)]],
                            add=True)
            return carry

        lax.fori_loop(0, N_CHUNK, body, 0)
        plsc.subcore_barrier()
        sl = pl.ds(s * ROWS_PER_TILE, ROWS_PER_TILE)
        pltpu.sync_copy(acc_sh.at[sl], out_h.at[c].at[sl])

    return k(table, src_flat, dst3, zeros_hbm)


def _sc_degree(dst3, ones_hbm, zeros_hbm):
    mesh = plsc.VectorSubcoreMesh(core_axis_name="c", subcore_axis_name="s")

    @functools.partial(
        pl.kernel,
        out_type=jax.ShapeDtypeStruct((2, N_PAD, FEAT), jnp.float32),
        mesh=mesh,
        scratch_types=[
            pltpu.VMEM((EPT,), jnp.int32),
            pltpu.VMEM((CHUNK, FEAT), jnp.float32),
            pltpu.VMEM_SHARED((N_PAD, FEAT), jnp.float32),
            pltpu.SemaphoreType.DMA,
            pltpu.SemaphoreType.DMA,
        ],
    )
    def k(dst_h, ones_h, zeros_h, out_h, dst_v, ones_v, acc_sh, sem_a, sem_b):
        c = lax.axis_index("c")
        s = lax.axis_index("s")
        w = c * 16 + s
        pltpu.sync_copy(dst_h.at[pl.ds(w * EPT, EPT)], dst_v)
        pltpu.sync_copy(ones_h, ones_v)
        pltpu.sync_copy(zeros_h, acc_sh.at[pl.ds(s * ROWS_PER_TILE, ROWS_PER_TILE)])
        plsc.subcore_barrier()

        def body(p, carry):
            pltpu.sync_copy(ones_v, acc_sh.at[dst_v.at[pl.ds(p * CHUNK, CHUNK)]],
                            add=True)
            return carry

        lax.fori_loop(0, N_CHUNK, body, 0)
        plsc.subcore_barrier()
        sl = pl.ds(s * ROWS_PER_TILE, ROWS_PER_TILE)
        pltpu.sync_copy(acc_sh.at[sl], out_h.at[c].at[sl])

    return k(dst3, ones_hbm, zeros_hbm)


_R = 1000


def _part_specs():
    return [
        pl.BlockSpec((1, _R, FEAT), lambda i: (0, i, 0)),
        pl.BlockSpec((1, _R, FEAT), lambda i: (1, i, 0)),
    ]


def _tc_encode1(x, W1, degp):
    def body(x_r, w_r, d0_r, d1_r, h_r, dis_r):
        deg = 1.0 + d0_r[0][:, 0:1] + d1_r[0][:, 0:1]
        dis = lax.rsqrt(deg)
        h = jnp.dot(x_r[...], w_r[...], preferred_element_type=jnp.float32,
                    precision=lax.Precision.HIGHEST)
        h_r[...] = h * dis
        dis_r[...] = dis

    return pl.pallas_call(
        body,
        grid=(N_NODES // _R,),
        in_specs=[
            pl.BlockSpec((_R, 128), lambda i: (i, 0)),
            pl.BlockSpec((128, 128), lambda i: (0, 0)),
        ] + _part_specs(),
        out_specs=[
            pl.BlockSpec((_R, 128), lambda i: (i, 0)),
            pl.BlockSpec((_R, 1), lambda i: (i, 0)),
        ],
        out_shape=[
            jax.ShapeDtypeStruct((N_NODES, 128), jnp.float32),
            jax.ShapeDtypeStruct((N_NODES, 1), jnp.float32),
        ],
    )(x, W1, degp, degp)


def _tc_encode2(aggp, h1p, dis, b1, W2):
    def body(a0_r, a1_r, h1p_r, dis_r, b1_r, w2_r, o_r):
        t = (a0_r[0] + a1_r[0] + h1p_r[...]) * dis_r[...] + b1_r[...]
        o = jnp.maximum(t, 0.0)
        h2 = jnp.dot(o, w2_r[...], preferred_element_type=jnp.float32,
                     precision=lax.Precision.HIGHEST)
        o_r[...] = jnp.concatenate(
            [h2 * dis_r[...], jnp.zeros((h2.shape[0], 64), jnp.float32)], axis=1)

    return pl.pallas_call(
        body,
        grid=(N_NODES // _R,),
        in_specs=_part_specs() + [
            pl.BlockSpec((_R, 128), lambda i: (i, 0)),
            pl.BlockSpec((_R, 1), lambda i: (i, 0)),
            pl.BlockSpec((1, 128), lambda i: (0, 0)),
            pl.BlockSpec((128, 64), lambda i: (0, 0)),
        ],
        out_specs=pl.BlockSpec((_R, 128), lambda i: (i, 0)),
        out_shape=jax.ShapeDtypeStruct((N_NODES, 128), jnp.float32),
    )(aggp, aggp, h1p, dis, b1, W2)


def _tc_decode(aggp, h2p, dis, b2p, Wdp, bd):
    def body(a0_r, a1_r, h2p_r, dis_r, b2_r, wd_r, bd_r, o_r):
        z = (a0_r[0] + a1_r[0] + h2p_r[...]) * dis_r[...] + b2_r[...]
        dec = jnp.dot(z, wd_r[...], preferred_element_type=jnp.float32,
                      precision=lax.Precision.HIGHEST) + bd_r[...]
        o_r[...] = 1.0 / (1.0 + jnp.exp(-dec))

    return pl.pallas_call(
        body,
        grid=(N_NODES // _R,),
        in_specs=_part_specs() + [
            pl.BlockSpec((_R, 128), lambda i: (i, 0)),
            pl.BlockSpec((_R, 1), lambda i: (i, 0)),
            pl.BlockSpec((1, 128), lambda i: (0, 0)),
            pl.BlockSpec((128, 128), lambda i: (0, 0)),
            pl.BlockSpec((1, 128), lambda i: (0, 0)),
        ],
        out_specs=pl.BlockSpec((_R, 128), lambda i: (i, 0)),
        out_shape=jax.ShapeDtypeStruct((N_NODES, 128), jnp.float32),
    )(aggp, aggp, h2p, dis, b2p, Wdp, bd)


def kernel(x, edge_index, W1, b1, W2, b2, Wd, bd):
    srcw = edge_index[0].reshape(NW, EPW)
    dstw = edge_index[1].reshape(NW, EPW)
    pad_s = jnp.zeros((NW, PADE), jnp.int32)
    pad_d = jnp.broadcast_to(N_NODES + jnp.arange(PADE, dtype=jnp.int32),
                             (NW, PADE))
    src_flat = jnp.concatenate([srcw, pad_s], axis=1).reshape(NW * EPT)
    dst_flat = jnp.concatenate([dstw, pad_d], axis=1).reshape(NW * EPT)

    zeros128 = jnp.zeros((ROWS_PER_TILE, FEAT), jnp.float32)
    ones128 = jnp.ones((CHUNK, FEAT), jnp.float32)
    degp = _sc_degree(dst_flat, ones128, zeros128)

    h1p, dis = _tc_encode1(x, W1, degp)

    agg1 = _sc_edge_agg(h1p, src_flat, dst_flat, zeros128)
    h2p = _tc_encode2(agg1, h1p, dis, b1.reshape(1, 128), W2)

    agg2 = _sc_edge_agg(h2p, src_flat, dst_flat, zeros128)
    b2p = jnp.zeros((1, 128), jnp.float32).at[0, :64].set(b2)
    Wdp = jnp.zeros((128, 128), jnp.float32).at[:64, :].set(Wd)
    return _tc_decode(agg2, h2p, dis, b2p, Wdp, bd.reshape(1, 128))


# speedup vs baseline: 2.2913x; 1.8950x over previous
import functools

import jax
import jax.numpy as jnp
from jax import lax
from jax.experimental import pallas as pl
from jax.experimental.pallas import tpu as pltpu
from jax.experimental.pallas import tpu_sc as plsc

N_NODES = 10000
N_PAD = 10240
N_EDGES = 320000
NW = 32
N_CHUNK = 50
CHUNK = 200
EPW = N_EDGES // NW
EPT = N_CHUNK * CHUNK
PADE = EPT - EPW
ROWS_PER_TILE = N_PAD // 16
FEAT = 128


def _sc_edge_agg(table, src_flat, dst3, zeros_hbm):
    mesh = plsc.VectorSubcoreMesh(core_axis_name="c", subcore_axis_name="s")

    @functools.partial(
        pl.kernel,
        out_type=jax.ShapeDtypeStruct((2, N_PAD, FEAT), jnp.float32),
        mesh=mesh,
        scratch_types=[
            pltpu.VMEM((EPT,), jnp.int32),
            pltpu.VMEM((EPT,), jnp.int32),
            pltpu.VMEM((CHUNK, FEAT), jnp.float32),
            pltpu.VMEM_SHARED((N_PAD, FEAT), jnp.float32),
            pltpu.SemaphoreType.DMA,
            pltpu.SemaphoreType.DMA,
        ],
    )
    def k(table_h, src_h, dst_h, zeros_h, out_h,
          src_v, dst_v, rows_v, acc_sh, sem_a, sem_b):
        c = lax.axis_index("c")
        s = lax.axis_index("s")
        w = c * 16 + s
        pltpu.sync_copy(src_h.at[pl.ds(w * EPT, EPT)], src_v)
        pltpu.sync_copy(dst_h.at[pl.ds(w * EPT, EPT)], dst_v)
        pltpu.sync_copy(zeros_h, acc_sh.at[pl.ds(s * ROWS_PER_TILE, ROWS_PER_TILE)])
        plsc.subcore_barrier()

        def body(p, carry):
            src_sl = src_v.at[pl.ds(p * CHUNK, CHUNK)]
            pltpu.async_copy(table_h.at[src_sl], rows_v, sem_a).wait()
            pltpu.sync_copy(rows_v, acc_sh.at[dst_v.at[pl.ds(p * CHUNK, CHUNK)]],
                            add=True)
            return carry

        lax.fori_loop(0, N_CHUNK, body, 0)
        plsc.subcore_barrier()
        sl = pl.ds(s * ROWS_PER_TILE, ROWS_PER_TILE)
        pltpu.sync_copy(acc_sh.at[sl], out_h.at[c].at[sl])

    return k(table, src_flat, dst3, zeros_hbm)


def _sc_degree(dst3, ones_hbm, zeros_hbm):
    mesh = plsc.VectorSubcoreMesh(core_axis_name="c", subcore_axis_name="s")

    @functools.partial(
        pl.kernel,
        out_type=jax.ShapeDtypeStruct((2, N_PAD, FEAT), jnp.float32),
        mesh=mesh,
        scratch_types=[
            pltpu.VMEM((EPT,), jnp.int32),
            pltpu.VMEM((CHUNK, FEAT), jnp.float32),
            pltpu.VMEM_SHARED((N_PAD, FEAT), jnp.float32),
            pltpu.SemaphoreType.DMA,
            pltpu.SemaphoreType.DMA,
        ],
    )
    def k(dst_h, ones_h, zeros_h, out_h, dst_v, ones_v, acc_sh, sem_a, sem_b):
        c = lax.axis_index("c")
        s = lax.axis_index("s")
        w = c * 16 + s
        pltpu.sync_copy(dst_h.at[pl.ds(w * EPT, EPT)], dst_v)
        pltpu.sync_copy(ones_h, ones_v)
        pltpu.sync_copy(zeros_h, acc_sh.at[pl.ds(s * ROWS_PER_TILE, ROWS_PER_TILE)])
        plsc.subcore_barrier()

        def body(p, carry):
            pltpu.sync_copy(ones_v, acc_sh.at[dst_v.at[pl.ds(p * CHUNK, CHUNK)]],
                            add=True)
            return carry

        lax.fori_loop(0, N_CHUNK, body, 0)
        plsc.subcore_barrier()
        sl = pl.ds(s * ROWS_PER_TILE, ROWS_PER_TILE)
        pltpu.sync_copy(acc_sh.at[sl], out_h.at[c].at[sl])

    return k(dst3, ones_hbm, zeros_hbm)


_R = 1000


def _part_specs():
    return [
        pl.BlockSpec((1, _R, FEAT), lambda i: (0, i, 0)),
        pl.BlockSpec((1, _R, FEAT), lambda i: (1, i, 0)),
    ]


def _tc_encode1(x, W1, degp):
    def body(x_r, w_r, d0_r, d1_r, h_r, dis_r):
        deg = 1.0 + d0_r[0][:, 0:1] + d1_r[0][:, 0:1]
        dis = lax.rsqrt(deg)
        h = jnp.dot(x_r[...], w_r[...], preferred_element_type=jnp.float32,
                    precision=lax.Precision.HIGHEST)
        h_r[...] = h * dis
        dis_r[...] = dis

    return pl.pallas_call(
        body,
        grid=(N_NODES // _R,),
        in_specs=[
            pl.BlockSpec((_R, 128), lambda i: (i, 0)),
            pl.BlockSpec((128, 128), lambda i: (0, 0)),
        ] + _part_specs(),
        out_specs=[
            pl.BlockSpec((_R, 128), lambda i: (i, 0)),
            pl.BlockSpec((_R, 1), lambda i: (i, 0)),
        ],
        out_shape=[
            jax.ShapeDtypeStruct((N_NODES, 128), jnp.float32),
            jax.ShapeDtypeStruct((N_NODES, 1), jnp.float32),
        ],
    )(x, W1, degp, degp)


def _tc_encode2(aggp, h1p, dis, b1, W2):
    def body(a0_r, a1_r, h1p_r, dis_r, b1_r, w2_r, o_r):
        t = (a0_r[0] + a1_r[0] + h1p_r[...]) * dis_r[...] + b1_r[...]
        o = jnp.maximum(t, 0.0)
        h2 = jnp.dot(o, w2_r[...], preferred_element_type=jnp.float32,
                     precision=lax.Precision.HIGHEST)
        o_r[...] = jnp.concatenate(
            [h2 * dis_r[...], jnp.zeros((h2.shape[0], 64), jnp.float32)], axis=1)

    return pl.pallas_call(
        body,
        grid=(N_NODES // _R,),
        in_specs=_part_specs() + [
            pl.BlockSpec((_R, 128), lambda i: (i, 0)),
            pl.BlockSpec((_R, 1), lambda i: (i, 0)),
            pl.BlockSpec((1, 128), lambda i: (0, 0)),
            pl.BlockSpec((128, 64), lambda i: (0, 0)),
        ],
        out_specs=pl.BlockSpec((_R, 128), lambda i: (i, 0)),
        out_shape=jax.ShapeDtypeStruct((N_NODES, 128), jnp.float32),
    )(aggp, aggp, h1p, dis, b1, W2)


def _tc_decode(aggp, h2p, dis, b2p, Wdp, bd):
    def body(a0_r, a1_r, h2p_r, dis_r, b2_r, wd_r, bd_r, o_r):
        z = (a0_r[0] + a1_r[0] + h2p_r[...]) * dis_r[...] + b2_r[...]
        dec = jnp.dot(z, wd_r[...], preferred_element_type=jnp.float32,
                      precision=lax.Precision.HIGHEST) + bd_r[...]
        o_r[...] = 1.0 / (1.0 + jnp.exp(-dec))

    return pl.pallas_call(
        body,
        grid=(N_NODES // _R,),
        in_specs=_part_specs() + [
            pl.BlockSpec((_R, 128), lambda i: (i, 0)),
            pl.BlockSpec((_R, 1), lambda i: (i, 0)),
            pl.BlockSpec((1, 128), lambda i: (0, 0)),
            pl.BlockSpec((128, 128), lambda i: (0, 0)),
            pl.BlockSpec((1, 128), lambda i: (0, 0)),
        ],
        out_specs=pl.BlockSpec((_R, 128), lambda i: (i, 0)),
        out_shape=jax.ShapeDtypeStruct((N_NODES, 128), jnp.float32),
    )(aggp, aggp, h2p, dis, b2p, Wdp, bd)


def kernel(x, edge_index, W1, b1, W2, b2, Wd, bd):
    srcw = edge_index[0].reshape(NW, EPW)
    dstw = edge_index[1].reshape(NW, EPW)
    pad_s = jnp.zeros((NW, PADE), jnp.int32)
    pad_d = jnp.broadcast_to(N_NODES + jnp.arange(PADE, dtype=jnp.int32),
                             (NW, PADE))
    if PADE:
        srcw = jnp.concatenate([srcw, pad_s], axis=1)
        dstw = jnp.concatenate([dstw, pad_d], axis=1)
    src_flat = srcw.reshape(NW * EPT)
    dst_flat = dstw.reshape(NW * EPT)

    zeros128 = jnp.zeros((ROWS_PER_TILE, FEAT), jnp.float32)
    ones128 = jnp.ones((CHUNK, FEAT), jnp.float32)
    degp = _sc_degree(dst_flat, ones128, zeros128)

    h1p, dis = _tc_encode1(x, W1, degp)

    agg1 = _sc_edge_agg(h1p, src_flat, dst_flat, zeros128)
    h2p = _tc_encode2(agg1, h1p, dis, b1.reshape(1, 128), W2)

    agg2 = _sc_edge_agg(h2p, src_flat, dst_flat, zeros128)
    b2p = jnp.zeros((1, 128), jnp.float32).at[0, :64].set(b2)
    Wdp = jnp.zeros((128, 128), jnp.float32).at[:64, :].set(Wd)
    return _tc_decode(agg2, h2p, dis, b2p, Wdp, bd.reshape(1, 128))


# speedup vs baseline: 2.7232x; 1.1885x over previous
import functools

import jax
import jax.numpy as jnp
from jax import lax
from jax.experimental import pallas as pl
from jax.experimental.pallas import tpu as pltpu
from jax.experimental.pallas import tpu_sc as plsc

N_NODES = 10000
N_PAD = 10240
N_EDGES = 320000
NW = 32
N_CHUNK = 125
CHUNK = 80
EPW = N_EDGES // NW
EPT = N_CHUNK * CHUNK
PADE = EPT - EPW
ROWS_PER_TILE = N_PAD // 16
FEAT = 128


def _sc_edge_agg(table, src_flat, dst3, zeros_hbm):
    mesh = plsc.VectorSubcoreMesh(core_axis_name="c", subcore_axis_name="s")

    @functools.partial(
        pl.kernel,
        out_type=jax.ShapeDtypeStruct((2, N_PAD, FEAT), jnp.float32),
        mesh=mesh,
        scratch_types=[
            pltpu.VMEM((EPT,), jnp.int32),
            pltpu.VMEM((EPT,), jnp.int32),
            pltpu.VMEM((CHUNK, FEAT), jnp.float32),
            pltpu.VMEM((CHUNK, FEAT), jnp.float32),
            pltpu.VMEM_SHARED((N_PAD, FEAT), jnp.float32),
            pltpu.SemaphoreType.DMA,
            pltpu.SemaphoreType.DMA,
        ],
    )
    def k(table_h, src_h, dst_h, zeros_h, out_h,
          src_v, dst_v, rows_a, rows_b, acc_sh, sem_a, sem_b):
        c = lax.axis_index("c")
        s = lax.axis_index("s")
        w = c * 16 + s
        pltpu.sync_copy(src_h.at[pl.ds(w * EPT, EPT)], src_v)
        pltpu.sync_copy(dst_h.at[pl.ds(w * EPT, EPT)], dst_v)
        pltpu.sync_copy(zeros_h, acc_sh.at[pl.ds(s * ROWS_PER_TILE, ROWS_PER_TILE)])
        plsc.subcore_barrier()

        def src_of(j):
            return src_v.at[pl.ds(j * CHUNK, CHUNK)]

        def dst_of(j):
            return dst_v.at[pl.ds(j * CHUNK, CHUNK)]

        pltpu.async_copy(table_h.at[src_of(0)], rows_a, sem_a)

        def pair(i, carry):
            j0 = 2 * i
            gb = pltpu.async_copy(table_h.at[src_of(j0 + 1)], rows_b, sem_b)
            pltpu.make_async_copy(table_h.at[src_of(j0)], rows_a, sem_a).wait()
            pltpu.sync_copy(rows_a, acc_sh.at[dst_of(j0)], add=True)

            @pl.when(j0 + 2 < N_CHUNK)
            def _():
                pltpu.async_copy(table_h.at[src_of(j0 + 2)], rows_a, sem_a)

            gb.wait()
            pltpu.sync_copy(rows_b, acc_sh.at[dst_of(j0 + 1)], add=True)
            return carry

        lax.fori_loop(0, N_CHUNK // 2, pair, 0)
        if N_CHUNK % 2:
            last = N_CHUNK - 1
            pltpu.make_async_copy(table_h.at[src_of(last)], rows_a, sem_a).wait()
            pltpu.sync_copy(rows_a, acc_sh.at[dst_of(last)], add=True)
        plsc.subcore_barrier()
        sl = pl.ds(s * ROWS_PER_TILE, ROWS_PER_TILE)
        pltpu.sync_copy(acc_sh.at[sl], out_h.at[c].at[sl])

    return k(table, src_flat, dst3, zeros_hbm)


def _sc_degree(dst3, ones_hbm, zeros_hbm):
    mesh = plsc.VectorSubcoreMesh(core_axis_name="c", subcore_axis_name="s")

    @functools.partial(
        pl.kernel,
        out_type=jax.ShapeDtypeStruct((2, N_PAD, FEAT), jnp.float32),
        mesh=mesh,
        scratch_types=[
            pltpu.VMEM((EPT,), jnp.int32),
            pltpu.VMEM((CHUNK, FEAT), jnp.float32),
            pltpu.VMEM_SHARED((N_PAD, FEAT), jnp.float32),
            pltpu.SemaphoreType.DMA,
            pltpu.SemaphoreType.DMA,
        ],
    )
    def k(dst_h, ones_h, zeros_h, out_h, dst_v, ones_v, acc_sh, sem_a, sem_b):
        c = lax.axis_index("c")
        s = lax.axis_index("s")
        w = c * 16 + s
        pltpu.sync_copy(dst_h.at[pl.ds(w * EPT, EPT)], dst_v)
        pltpu.sync_copy(ones_h, ones_v)
        pltpu.sync_copy(zeros_h, acc_sh.at[pl.ds(s * ROWS_PER_TILE, ROWS_PER_TILE)])
        plsc.subcore_barrier()

        def body(p, carry):
            pltpu.sync_copy(ones_v, acc_sh.at[dst_v.at[pl.ds(p * CHUNK, CHUNK)]],
                            add=True)
            return carry

        lax.fori_loop(0, N_CHUNK, body, 0)
        plsc.subcore_barrier()
        sl = pl.ds(s * ROWS_PER_TILE, ROWS_PER_TILE)
        pltpu.sync_copy(acc_sh.at[sl], out_h.at[c].at[sl])

    return k(dst3, ones_hbm, zeros_hbm)


_R = 1000


def _part_specs():
    return [
        pl.BlockSpec((1, _R, FEAT), lambda i: (0, i, 0)),
        pl.BlockSpec((1, _R, FEAT), lambda i: (1, i, 0)),
    ]


def _tc_encode1(x, W1, degp):
    def body(x_r, w_r, d0_r, d1_r, h_r, dis_r):
        deg = 1.0 + d0_r[0][:, 0:1] + d1_r[0][:, 0:1]
        dis = lax.rsqrt(deg)
        h = jnp.dot(x_r[...], w_r[...], preferred_element_type=jnp.float32,
                    precision=lax.Precision.HIGHEST)
        h_r[...] = h * dis
        dis_r[...] = dis

    return pl.pallas_call(
        body,
        grid=(N_NODES // _R,),
        in_specs=[
            pl.BlockSpec((_R, 128), lambda i: (i, 0)),
            pl.BlockSpec((128, 128), lambda i: (0, 0)),
        ] + _part_specs(),
        out_specs=[
            pl.BlockSpec((_R, 128), lambda i: (i, 0)),
            pl.BlockSpec((_R, 1), lambda i: (i, 0)),
        ],
        out_shape=[
            jax.ShapeDtypeStruct((N_NODES, 128), jnp.float32),
            jax.ShapeDtypeStruct((N_NODES, 1), jnp.float32),
        ],
    )(x, W1, degp, degp)


def _tc_encode2(aggp, h1p, dis, b1, W2):
    def body(a0_r, a1_r, h1p_r, dis_r, b1_r, w2_r, o_r):
        t = (a0_r[0] + a1_r[0] + h1p_r[...]) * dis_r[...] + b1_r[...]
        o = jnp.maximum(t, 0.0)
        h2 = jnp.dot(o, w2_r[...], preferred_element_type=jnp.float32,
                     precision=lax.Precision.HIGHEST)
        o_r[...] = jnp.concatenate(
            [h2 * dis_r[...], jnp.zeros((h2.shape[0], 64), jnp.float32)], axis=1)

    return pl.pallas_call(
        body,
        grid=(N_NODES // _R,),
        in_specs=_part_specs() + [
            pl.BlockSpec((_R, 128), lambda i: (i, 0)),
            pl.BlockSpec((_R, 1), lambda i: (i, 0)),
            pl.BlockSpec((1, 128), lambda i: (0, 0)),
            pl.BlockSpec((128, 64), lambda i: (0, 0)),
        ],
        out_specs=pl.BlockSpec((_R, 128), lambda i: (i, 0)),
        out_shape=jax.ShapeDtypeStruct((N_NODES, 128), jnp.float32),
    )(aggp, aggp, h1p, dis, b1, W2)


def _tc_decode(aggp, h2p, dis, b2p, Wdp, bd):
    def body(a0_r, a1_r, h2p_r, dis_r, b2_r, wd_r, bd_r, o_r):
        z = (a0_r[0] + a1_r[0] + h2p_r[...]) * dis_r[...] + b2_r[...]
        dec = jnp.dot(z, wd_r[...], preferred_element_type=jnp.float32,
                      precision=lax.Precision.HIGHEST) + bd_r[...]
        o_r[...] = 1.0 / (1.0 + jnp.exp(-dec))

    return pl.pallas_call(
        body,
        grid=(N_NODES // _R,),
        in_specs=_part_specs() + [
            pl.BlockSpec((_R, 128), lambda i: (i, 0)),
            pl.BlockSpec((_R, 1), lambda i: (i, 0)),
            pl.BlockSpec((1, 128), lambda i: (0, 0)),
            pl.BlockSpec((128, 128), lambda i: (0, 0)),
            pl.BlockSpec((1, 128), lambda i: (0, 0)),
        ],
        out_specs=pl.BlockSpec((_R, 128), lambda i: (i, 0)),
        out_shape=jax.ShapeDtypeStruct((N_NODES, 128), jnp.float32),
    )(aggp, aggp, h2p, dis, b2p, Wdp, bd)


def kernel(x, edge_index, W1, b1, W2, b2, Wd, bd):
    srcw = edge_index[0].reshape(NW, EPW)
    dstw = edge_index[1].reshape(NW, EPW)
    pad_s = jnp.zeros((NW, PADE), jnp.int32)
    pad_d = jnp.broadcast_to(N_NODES + jnp.arange(PADE, dtype=jnp.int32),
                             (NW, PADE))
    if PADE:
        srcw = jnp.concatenate([srcw, pad_s], axis=1)
        dstw = jnp.concatenate([dstw, pad_d], axis=1)
    src_flat = srcw.reshape(NW * EPT)
    dst_flat = dstw.reshape(NW * EPT)

    zeros128 = jnp.zeros((ROWS_PER_TILE, FEAT), jnp.float32)
    ones128 = jnp.ones((CHUNK, FEAT), jnp.float32)
    degp = _sc_degree(dst_flat, ones128, zeros128)

    h1p, dis = _tc_encode1(x, W1, degp)

    agg1 = _sc_edge_agg(h1p, src_flat, dst_flat, zeros128)
    h2p = _tc_encode2(agg1, h1p, dis, b1.reshape(1, 128), W2)

    agg2 = _sc_edge_agg(h2p, src_flat, dst_flat, zeros128)
    b2p = jnp.zeros((1, 128), jnp.float32).at[0, :64].set(b2)
    Wdp = jnp.zeros((128, 128), jnp.float32).at[:64, :].set(Wd)
    return _tc_decode(agg2, h2p, dis, b2p, Wdp, bd.reshape(1, 128))


# speedup vs baseline: 3.0868x; 1.1335x over previous
import functools

import jax
import jax.numpy as jnp
from jax import lax
from jax.experimental import pallas as pl
from jax.experimental.pallas import tpu as pltpu
from jax.experimental.pallas import tpu_sc as plsc

N_NODES = 10000
N_PAD = 10240
N_EDGES = 320000
NW = 32
N_CHUNK = 125
CHUNK = 80
EPW = N_EDGES // NW
EPT = N_CHUNK * CHUNK
PADE = EPT - EPW
ROWS_PER_TILE = N_PAD // 16
FEAT = 128


def _sc_edge_agg(table, src_flat, dst3, zeros_hbm):
    mesh = plsc.VectorSubcoreMesh(core_axis_name="c", subcore_axis_name="s")

    @functools.partial(
        pl.kernel,
        out_type=jax.ShapeDtypeStruct((2, N_PAD, FEAT), jnp.float32),
        mesh=mesh,
        scratch_types=[
            pltpu.VMEM((EPT,), jnp.int32),
            pltpu.VMEM((EPT,), jnp.int32),
            pltpu.VMEM((CHUNK, FEAT), jnp.float32),
            pltpu.VMEM((CHUNK, FEAT), jnp.float32),
            pltpu.VMEM_SHARED((N_PAD, FEAT), jnp.float32),
            pltpu.SemaphoreType.DMA,
            pltpu.SemaphoreType.DMA,
        ],
    )
    def k(table_h, src_h, dst_h, zeros_h, out_h,
          src_v, dst_v, rows_a, rows_b, acc_sh, sem_a, sem_b):
        c = lax.axis_index("c")
        s = lax.axis_index("s")
        w = c * 16 + s
        pltpu.sync_copy(src_h.at[pl.ds(w * EPT, EPT)], src_v)
        pltpu.sync_copy(dst_h.at[pl.ds(w * EPT, EPT)], dst_v)
        pltpu.sync_copy(zeros_h, acc_sh.at[pl.ds(s * ROWS_PER_TILE, ROWS_PER_TILE)])
        plsc.subcore_barrier()

        def src_of(j):
            return src_v.at[pl.ds(j * CHUNK, CHUNK)]

        def dst_of(j):
            return dst_v.at[pl.ds(j * CHUNK, CHUNK)]

        pltpu.async_copy(table_h.at[src_of(0)], rows_a, sem_a)

        def pair(i, carry):
            j0 = 2 * i
            gb = pltpu.async_copy(table_h.at[src_of(j0 + 1)], rows_b, sem_b)
            pltpu.make_async_copy(table_h.at[src_of(j0)], rows_a, sem_a).wait()
            pltpu.sync_copy(rows_a, acc_sh.at[dst_of(j0)], add=True)

            @pl.when(j0 + 2 < N_CHUNK)
            def _():
                pltpu.async_copy(table_h.at[src_of(j0 + 2)], rows_a, sem_a)

            gb.wait()
            pltpu.sync_copy(rows_b, acc_sh.at[dst_of(j0 + 1)], add=True)
            return carry

        lax.fori_loop(0, N_CHUNK // 2, pair, 0)
        if N_CHUNK % 2:
            last = N_CHUNK - 1
            pltpu.make_async_copy(table_h.at[src_of(last)], rows_a, sem_a).wait()
            pltpu.sync_copy(rows_a, acc_sh.at[dst_of(last)], add=True)
        plsc.subcore_barrier()
        sl = pl.ds(s * ROWS_PER_TILE, ROWS_PER_TILE)
        pltpu.sync_copy(acc_sh.at[sl], out_h.at[c].at[sl])

    return k(table, src_flat, dst3, zeros_hbm)


def _sc_degree(dst_flat, ones_hbm, zeros_hbm):
    mesh = plsc.VectorSubcoreMesh(core_axis_name="c", subcore_axis_name="s")

    @functools.partial(
        pl.kernel,
        out_type=jax.ShapeDtypeStruct((2 * N_PAD,), jnp.float32),
        mesh=mesh,
        scratch_types=[
            pltpu.VMEM((EPT,), jnp.int32),
            pltpu.VMEM((CHUNK,), jnp.float32),
            pltpu.VMEM_SHARED((N_PAD,), jnp.float32),
            pltpu.SemaphoreType.DMA,
        ],
    )
    def k(dst_h, ones_h, zeros_h, out_h, dst_v, ones_v, acc_sh, sem_a):
        c = lax.axis_index("c")
        s = lax.axis_index("s")
        w = c * 16 + s
        pltpu.sync_copy(dst_h.at[pl.ds(w * EPT, EPT)], dst_v)
        pltpu.sync_copy(ones_h, ones_v)
        pltpu.sync_copy(zeros_h, acc_sh.at[pl.ds(s * ROWS_PER_TILE, ROWS_PER_TILE)])
        plsc.subcore_barrier()

        def body(p, carry):
            pltpu.sync_copy(ones_v, acc_sh.at[dst_v.at[pl.ds(p * CHUNK, CHUNK)]],
                            add=True)
            return carry

        lax.fori_loop(0, N_CHUNK, body, 0)
        plsc.subcore_barrier()
        sl = pl.ds(s * ROWS_PER_TILE, ROWS_PER_TILE)
        pltpu.sync_copy(acc_sh.at[sl],
                        out_h.at[pl.ds(c * N_PAD + s * ROWS_PER_TILE,
                                       ROWS_PER_TILE)])

    return k(dst_flat, ones_hbm, zeros_hbm)


_R = 1000


def _part_specs():
    return [
        pl.BlockSpec((1, _R, FEAT), lambda i: (0, i, 0)),
        pl.BlockSpec((1, _R, FEAT), lambda i: (1, i, 0)),
    ]


def _tc_encode1(x, W1, p0, p1):
    def body(x_r, w_r, d0_r, d1_r, h_r, dis_r):
        deg = 1.0 + d0_r[...] + d1_r[...]
        dis = lax.rsqrt(deg)
        h = jnp.dot(x_r[...], w_r[...], preferred_element_type=jnp.float32,
                    precision=lax.Precision.HIGHEST)
        h_r[...] = h * dis
        dis_r[...] = dis

    return pl.pallas_call(
        body,
        grid=(N_NODES // _R,),
        in_specs=[
            pl.BlockSpec((_R, 128), lambda i: (i, 0)),
            pl.BlockSpec((128, 128), lambda i: (0, 0)),
            pl.BlockSpec((_R, 1), lambda i: (i, 0)),
            pl.BlockSpec((_R, 1), lambda i: (i, 0)),
        ],
        out_specs=[
            pl.BlockSpec((_R, 128), lambda i: (i, 0)),
            pl.BlockSpec((_R, 1), lambda i: (i, 0)),
        ],
        out_shape=[
            jax.ShapeDtypeStruct((N_NODES, 128), jnp.float32),
            jax.ShapeDtypeStruct((N_NODES, 1), jnp.float32),
        ],
    )(x, W1, p0, p1)


def _tc_encode2(aggp, h1p, dis, b1, W2):
    def body(a0_r, a1_r, h1p_r, dis_r, b1_r, w2_r, o_r):
        t = (a0_r[0] + a1_r[0] + h1p_r[...]) * dis_r[...] + b1_r[...]
        o = jnp.maximum(t, 0.0)
        h2 = jnp.dot(o, w2_r[...], preferred_element_type=jnp.float32,
                     precision=lax.Precision.HIGHEST)
        o_r[...] = jnp.concatenate(
            [h2 * dis_r[...], jnp.zeros((h2.shape[0], 64), jnp.float32)], axis=1)

    return pl.pallas_call(
        body,
        grid=(N_NODES // _R,),
        in_specs=_part_specs() + [
            pl.BlockSpec((_R, 128), lambda i: (i, 0)),
            pl.BlockSpec((_R, 1), lambda i: (i, 0)),
            pl.BlockSpec((1, 128), lambda i: (0, 0)),
            pl.BlockSpec((128, 64), lambda i: (0, 0)),
        ],
        out_specs=pl.BlockSpec((_R, 128), lambda i: (i, 0)),
        out_shape=jax.ShapeDtypeStruct((N_NODES, 128), jnp.float32),
    )(aggp, aggp, h1p, dis, b1, W2)


def _tc_decode(aggp, h2p, dis, b2p, Wdp, bd):
    def body(a0_r, a1_r, h2p_r, dis_r, b2_r, wd_r, bd_r, o_r):
        z = (a0_r[0] + a1_r[0] + h2p_r[...]) * dis_r[...] + b2_r[...]
        dec = jnp.dot(z, wd_r[...], preferred_element_type=jnp.float32,
                      precision=lax.Precision.HIGHEST) + bd_r[...]
        o_r[...] = 1.0 / (1.0 + jnp.exp(-dec))

    return pl.pallas_call(
        body,
        grid=(N_NODES // _R,),
        in_specs=_part_specs() + [
            pl.BlockSpec((_R, 128), lambda i: (i, 0)),
            pl.BlockSpec((_R, 1), lambda i: (i, 0)),
            pl.BlockSpec((1, 128), lambda i: (0, 0)),
            pl.BlockSpec((128, 128), lambda i: (0, 0)),
            pl.BlockSpec((1, 128), lambda i: (0, 0)),
        ],
        out_specs=pl.BlockSpec((_R, 128), lambda i: (i, 0)),
        out_shape=jax.ShapeDtypeStruct((N_NODES, 128), jnp.float32),
    )(aggp, aggp, h2p, dis, b2p, Wdp, bd)


def kernel(x, edge_index, W1, b1, W2, b2, Wd, bd):
    srcw = edge_index[0].reshape(NW, EPW)
    dstw = edge_index[1].reshape(NW, EPW)
    pad_s = jnp.zeros((NW, PADE), jnp.int32)
    pad_d = jnp.broadcast_to(N_NODES + jnp.arange(PADE, dtype=jnp.int32),
                             (NW, PADE))
    if PADE:
        srcw = jnp.concatenate([srcw, pad_s], axis=1)
        dstw = jnp.concatenate([dstw, pad_d], axis=1)
    src_flat = srcw.reshape(NW * EPT)
    dst_flat = dstw.reshape(NW * EPT)

    zeros128 = jnp.zeros((ROWS_PER_TILE, FEAT), jnp.float32)
    zeros1 = jnp.zeros((ROWS_PER_TILE,), jnp.float32)
    ones1 = jnp.ones((CHUNK,), jnp.float32)
    degv = _sc_degree(dst_flat, ones1, zeros1)
    p0 = degv[:N_NODES, None]
    p1 = degv[N_PAD:N_PAD + N_NODES, None]

    h1p, dis = _tc_encode1(x, W1, p0, p1)

    agg1 = _sc_edge_agg(h1p, src_flat, dst_flat, zeros128)
    h2p = _tc_encode2(agg1, h1p, dis, b1.reshape(1, 128), W2)

    agg2 = _sc_edge_agg(h2p, src_flat, dst_flat, zeros128)
    b2p = jnp.zeros((1, 128), jnp.float32).at[0, :64].set(b2)
    Wdp = jnp.zeros((128, 128), jnp.float32).at[:64, :].set(Wd)
    return _tc_decode(agg2, h2p, dis, b2p, Wdp, bd.reshape(1, 128))
